# baseline (speedup 1.0000x reference)
import jax
import jax.numpy as jnp
from jax import lax
from jax.experimental import pallas as pl
from jax.experimental.pallas import tpu as pltpu
from jax.experimental.pallas import tpu_sc as plsc

NUM_CORES = 2
NUM_SUBCORES = 16
NUM_WORKERS = NUM_CORES * NUM_SUBCORES

NB = 16384
S = 50
B = NB * S
D = 64
BATCHES_PW = NB // NUM_WORKERS
BPW = B // NUM_WORKERS
CB = 16
CHUNK = CB * S
NCHUNKS = BATCHES_PW // CB
assert BATCHES_PW % CB == 0


def _gather_body(table_hbm, idx_hbm, out3_hbm, idx_v, rows, sem):
    wid = lax.axis_index("s") * NUM_CORES + lax.axis_index("c")
    base = wid * BPW
    bbase = wid * BATCHES_PW
    pltpu.sync_copy(idx_hbm.at[pl.ds(base, BPW)], idx_v)

    @pl.loop(0, NCHUNKS)
    def _(j):
        off = j * CHUNK
        pltpu.async_copy(
            table_hbm.at[idx_v.at[pl.ds(off, CHUNK)]], rows, sem
        ).wait()
        b0 = bbase + j * CB
        for b in range(CB):
            pltpu.sync_copy(rows.at[pl.ds(b * S, S)], out3_hbm.at[b0 + b])


@jax.jit
def _gather(table, idx_flat):
    mesh = plsc.VectorSubcoreMesh(
        core_axis_name="c",
        subcore_axis_name="s",
        num_cores=NUM_CORES,
        num_subcores=NUM_SUBCORES,
    )
    fn = pl.kernel(
        _gather_body,
        out_type=jax.ShapeDtypeStruct((NB, S, D), jnp.float32),
        mesh=mesh,
        compiler_params=pltpu.CompilerParams(use_tc_tiling_on_sc=False),
        scratch_types=[
            pltpu.VMEM((BPW,), jnp.int32),
            pltpu.VMEM((CHUNK, D), jnp.float32),
            pltpu.SemaphoreType.DMA,
        ],
    )
    return fn(table, idx_flat)


def kernel(x, table):
    idx_flat = x.reshape(-1).astype(jnp.int32)
    return _gather(table, idx_flat)


# speedup vs baseline: 1.0008x; 1.0008x over previous
import jax
import jax.numpy as jnp
from jax import lax
from jax.experimental import pallas as pl
from jax.experimental.pallas import tpu as pltpu
from jax.experimental.pallas import tpu_sc as plsc

NUM_CORES = 2
NUM_SUBCORES = 16
NUM_WORKERS = NUM_CORES * NUM_SUBCORES

NB = 16384
S = 50
B = NB * S
D = 64
BATCHES_PW = NB // NUM_WORKERS
BPW = B // NUM_WORKERS
CB = 16
CHUNK = CB * S
NCHUNKS = BATCHES_PW // CB
assert BATCHES_PW % CB == 0


def _gather_body(table_hbm, idx_hbm, out3_hbm, idx_v, rows, sem):
    wid = lax.axis_index("s") * NUM_CORES + lax.axis_index("c")
    base = wid * BPW
    bbase = wid * BATCHES_PW
    pltpu.sync_copy(idx_hbm.at[pl.ds(base, BPW)], idx_v)

    @pl.loop(0, NCHUNKS)
    def _(j):
        off = j * CHUNK
        pltpu.async_copy(
            table_hbm.at[idx_v.at[pl.ds(off, CHUNK)]], rows, sem
        ).wait()
        b0 = bbase + j * CB
        for b in range(CB):
            pltpu.sync_copy(rows.at[pl.ds(b * S, S)], out3_hbm.at[b0 + b])


@jax.jit
def _gather(table, idx_flat):
    mesh = plsc.VectorSubcoreMesh(
        core_axis_name="c",
        subcore_axis_name="s",
        num_cores=NUM_CORES,
        num_subcores=NUM_SUBCORES,
    )
    fn = pl.kernel(
        _gather_body,
        out_type=jax.ShapeDtypeStruct((NB, S, D), jnp.float32),
        mesh=mesh,
        compiler_params=pltpu.CompilerParams(use_tc_tiling_on_sc=False),
        scratch_types=[
            pltpu.VMEM((BPW,), jnp.int32),
            pltpu.VMEM((CHUNK, D), jnp.float32),
            pltpu.SemaphoreType.DMA,
        ],
    )
    return fn(table, idx_flat)


def kernel(x, table):
    idx_flat = x.reshape(-1).astype(jnp.int32)
    table_lin = lax.optimization_barrier(table.reshape(-1))
    return _gather(table_lin.reshape(table.shape), idx_flat)


# speedup vs baseline: 1.0277x; 1.0269x over previous
import jax
import jax.numpy as jnp
from jax import lax
from jax.experimental import pallas as pl
from jax.experimental.pallas import tpu as pltpu
from jax.experimental.pallas import tpu_sc as plsc

NUM_CORES = 2
NUM_SUBCORES = 16
NUM_WORKERS = NUM_CORES * NUM_SUBCORES

NB = 16384
S = 50
B = NB * S
D = 64
BATCHES_PW = NB // NUM_WORKERS
BPW = B // NUM_WORKERS
CB = 8
CHUNK = CB * S
NCHUNKS = BATCHES_PW // CB
assert BATCHES_PW % CB == 0 and NCHUNKS % 2 == 0


def _gather_body(table_hbm, idx_hbm, out3_hbm, idx_v, rows0, rows1, sem0, sem1):
    wid = lax.axis_index("s") * NUM_CORES + lax.axis_index("c")
    base = wid * BPW
    bbase = wid * BATCHES_PW
    pltpu.sync_copy(idx_hbm.at[pl.ds(base, BPW)], idx_v)

    def src(j):
        return table_hbm.at[idx_v.at[pl.ds(j * CHUNK, CHUNK)]]

    def write_chunk(j, rows):
        b0 = bbase + j * CB
        for b in range(CB):
            pltpu.sync_copy(rows.at[pl.ds(b * S, S)], out3_hbm.at[b0 + b])

    pltpu.async_copy(src(0), rows0, sem0)

    @pl.loop(0, NCHUNKS, step=2)
    def _(g):
        pltpu.make_async_copy(src(g), rows0, sem0).wait()
        pltpu.async_copy(src(g + 1), rows1, sem1)
        write_chunk(g, rows0)
        pltpu.make_async_copy(src(g + 1), rows1, sem1).wait()

        @pl.when(g + 2 < NCHUNKS)
        def _():
            pltpu.async_copy(src(g + 2), rows0, sem0)

        write_chunk(g + 1, rows1)


@jax.jit
def _gather(table, idx_flat):
    mesh = plsc.VectorSubcoreMesh(
        core_axis_name="c",
        subcore_axis_name="s",
        num_cores=NUM_CORES,
        num_subcores=NUM_SUBCORES,
    )
    fn = pl.kernel(
        _gather_body,
        out_type=jax.ShapeDtypeStruct((NB, S, D), jnp.float32),
        mesh=mesh,
        compiler_params=pltpu.CompilerParams(use_tc_tiling_on_sc=False),
        scratch_types=[
            pltpu.VMEM((BPW,), jnp.int32),
            pltpu.VMEM((CHUNK, D), jnp.float32),
            pltpu.VMEM((CHUNK, D), jnp.float32),
            pltpu.SemaphoreType.DMA,
            pltpu.SemaphoreType.DMA,
        ],
    )
    return fn(table, idx_flat)


def kernel(x, table):
    idx_flat = x.reshape(-1).astype(jnp.int32)
    table_lin = lax.optimization_barrier(table.reshape(-1))
    return _gather(table_lin.reshape(table.shape), idx_flat)


# speedup vs baseline: 1.0294x; 1.0017x over previous
import jax
import jax.numpy as jnp
from jax import lax
from jax.experimental import pallas as pl
from jax.experimental.pallas import tpu as pltpu
from jax.experimental.pallas import tpu_sc as plsc

NUM_CORES = 2
NUM_SUBCORES = 16
NUM_WORKERS = NUM_CORES * NUM_SUBCORES

NB = 16384
S = 50
B = NB * S
D = 64
BATCHES_PW = NB // NUM_WORKERS
BPW = B // NUM_WORKERS
CB = 16
CHUNK = CB * S
NCHUNKS = BATCHES_PW // CB
assert BATCHES_PW % CB == 0 and NCHUNKS % 2 == 0


def _gather_body(table_hbm, idx_hbm, out3_hbm, idx_v, rows0, rows1, sem0, sem1):
    wid = lax.axis_index("s") * NUM_CORES + lax.axis_index("c")
    base = wid * BPW
    bbase = wid * BATCHES_PW
    pltpu.sync_copy(idx_hbm.at[pl.ds(base, BPW)], idx_v)

    def src(j):
        return table_hbm.at[idx_v.at[pl.ds(j * CHUNK, CHUNK)]]

    def write_chunk(j, rows):
        b0 = bbase + j * CB
        for b in range(CB):
            pltpu.sync_copy(rows.at[pl.ds(b * S, S)], out3_hbm.at[b0 + b])

    pltpu.async_copy(src(0), rows0, sem0)

    @pl.loop(0, NCHUNKS, step=2)
    def _(g):
        pltpu.make_async_copy(src(g), rows0, sem0).wait()
        pltpu.async_copy(src(g + 1), rows1, sem1)
        write_chunk(g, rows0)
        pltpu.make_async_copy(src(g + 1), rows1, sem1).wait()

        @pl.when(g + 2 < NCHUNKS)
        def _():
            pltpu.async_copy(src(g + 2), rows0, sem0)

        write_chunk(g + 1, rows1)


@jax.jit
def _gather(table, idx_flat):
    mesh = plsc.VectorSubcoreMesh(
        core_axis_name="c",
        subcore_axis_name="s",
        num_cores=NUM_CORES,
        num_subcores=NUM_SUBCORES,
    )
    fn = pl.kernel(
        _gather_body,
        out_type=jax.ShapeDtypeStruct((NB, S, D), jnp.float32),
        mesh=mesh,
        compiler_params=pltpu.CompilerParams(use_tc_tiling_on_sc=False),
        scratch_types=[
            pltpu.VMEM((BPW,), jnp.int32),
            pltpu.VMEM((CHUNK, D), jnp.float32),
            pltpu.VMEM((CHUNK, D), jnp.float32),
            pltpu.SemaphoreType.DMA,
            pltpu.SemaphoreType.DMA,
        ],
    )
    return fn(table, idx_flat)


def kernel(x, table):
    idx_flat = x.reshape(-1).astype(jnp.int32)
    table_lin = lax.optimization_barrier(table.reshape(-1))
    return _gather(table_lin.reshape(table.shape), idx_flat)
